# baseline (speedup 1.0000x reference)
import functools

import jax
import jax.numpy as jnp
from jax import lax
from jax.experimental import pallas as pl
from jax.experimental.pallas import tpu as pltpu
from jax.experimental.pallas import tpu_sc as plsc

TOTAL = 32768
NSEG = 16
VOCAB = 17
VPAD = 32
D = 512
NC = 2
NS = 16
NW = NC * NS
LANES = 16
TPW = TOTAL // NW
NVEC = TPW // LANES
NBIN = NSEG * VPAD


def _bcast_lane(vec, i):
    idx = jnp.full((LANES, 1), i, jnp.int32)
    dn = lax.GatherDimensionNumbers(
        offset_dims=(), collapsed_slice_dims=(0,), start_index_map=(0,)
    )
    return lax.gather(
        vec, idx, dn, (1,), mode=lax.GatherScatterMode.PROMISE_IN_BOUNDS
    )


def _sc_hist_body(tok_hbm, cu_hbm, out_hbm, tok_v, cu_v, hist_v, red_v, sem):
    w = lax.axis_index("s") * NC + lax.axis_index("c")
    base = w * TPW
    pltpu.sync_copy(tok_hbm.at[pl.ds(base * 1, TPW)], tok_v)
    pltpu.sync_copy(cu_hbm, cu_v)

    zero16 = jnp.zeros((LANES,), jnp.float32)

    def _zero(i, carry):
        hist_v[pl.ds(pl.multiple_of(i * LANES, LANES), LANES)] = zero16
        return carry

    lax.fori_loop(0, LANES * NBIN // LANES, _zero, 0)

    cu_r = cu_v[pl.ds(0, LANES)]
    bounds = [_bcast_lane(cu_r, b) for b in range(1, NSEG)]
    iota16 = lax.iota(jnp.int32, LANES)
    lane_off = iota16 * NBIN
    ones16 = jnp.full((LANES,), 1.0, jnp.float32)

    def _scatter(j, carry):
        off = pl.multiple_of(j * LANES, LANES)
        tok = tok_v[pl.ds(off, LANES)]
        idx = base + j * LANES + iota16
        seg = jnp.zeros((LANES,), jnp.int32)
        for bc in bounds:
            seg = seg + (idx >= bc).astype(jnp.int32)
        sidx = lane_off + seg * VPAD + tok
        plsc.addupdate_scatter(hist_v, [sidx], ones16)
        return carry

    lax.fori_loop(0, NVEC, _scatter, 0)

    def _reduce(q, carry):
        qoff = pl.multiple_of(q * LANES, LANES)
        acc = hist_v[pl.ds(qoff, LANES)]
        for l in range(1, LANES):
            acc = acc + hist_v[pl.ds(l * NBIN + qoff, LANES)]
        red_v[pl.ds(qoff, LANES)] = acc
        return carry

    lax.fori_loop(0, NBIN // LANES, _reduce, 0)

    pltpu.sync_copy(red_v, out_hbm.at[w])


_sc_hist = functools.partial(
    pl.kernel,
    out_type=jax.ShapeDtypeStruct((NW, NBIN), jnp.float32),
    mesh=plsc.VectorSubcoreMesh(core_axis_name="c", subcore_axis_name="s"),
    compiler_params=pltpu.CompilerParams(needs_layout_passes=False),
    scratch_types=[
        pltpu.VMEM((TPW,), jnp.int32),
        pltpu.VMEM((VPAD,), jnp.int32),
        pltpu.VMEM((LANES * NBIN,), jnp.float32),
        pltpu.VMEM((NBIN,), jnp.float32),
        pltpu.SemaphoreType.DMA,
    ],
)(_sc_hist_body)


def _combine_body(part_ref, tab_ref, w1_ref, b1_ref, w2_ref, b2_ref, out_ref):
    hist = jnp.sum(part_ref[...], axis=0)
    counts = jnp.sum(hist, axis=1, keepdims=True)
    hn = hist / counts
    h = jax.lax.dot_general(
        tab_ref[...], w1_ref[...], (((1,), (1,)), ((), ())),
        preferred_element_type=jnp.float32,
    ) + b1_ref[...]
    g = 0.5 * h * (1.0 + jax.lax.erf(h * 0.7071067811865476))
    mo = jax.lax.dot_general(
        g, w2_ref[...], (((1,), (1,)), ((), ())),
        preferred_element_type=jnp.float32,
    ) + b2_ref[...]
    out_ref[...] = jnp.dot(hn, mo, preferred_element_type=jnp.float32)


def kernel(packed_tokens, cu_seq_lens, table, W1, b1, W2, b2):
    cu_pad = jnp.full((VPAD,), TOTAL, jnp.int32).at[: NSEG + 1].set(
        cu_seq_lens.astype(jnp.int32)
    )
    tab = jnp.zeros((VPAD, D), jnp.float32).at[:VOCAB].set(table)

    partials = _sc_hist(packed_tokens, cu_pad)
    part3d = partials.reshape(NW, NSEG, VPAD)

    out = pl.pallas_call(
        _combine_body,
        out_shape=jax.ShapeDtypeStruct((NSEG, D), jnp.float32),
    )(part3d, tab, W1, b1.reshape(1, D), W2, b2.reshape(1, D))
    return out


# speedup vs baseline: 1.0167x; 1.0167x over previous
import functools

import jax
import jax.numpy as jnp
from jax import lax
from jax.experimental import pallas as pl
from jax.experimental.pallas import tpu as pltpu
from jax.experimental.pallas import tpu_sc as plsc

TOTAL = 32768
NSEG = 16
VOCAB = 17
VPAD = 32
D = 512
NC = 2
NS = 16
NW = NC * NS
LANES = 16
TPW = TOTAL // NW
NVEC = TPW // LANES
NBIN = NSEG * VPAD


def _bcast_lane(vec, i):
    idx = jnp.full((LANES, 1), i, jnp.int32)
    dn = lax.GatherDimensionNumbers(
        offset_dims=(), collapsed_slice_dims=(0,), start_index_map=(0,)
    )
    return lax.gather(
        vec, idx, dn, (1,), mode=lax.GatherScatterMode.PROMISE_IN_BOUNDS
    )


def _sc_hist_body(tok_hbm, cu_hbm, out_hbm, tok_v, cu_v, hist_v, red_v, sem):
    w = lax.axis_index("s") * NC + lax.axis_index("c")
    base = w * TPW
    pltpu.sync_copy(tok_hbm.at[pl.ds(base * 1, TPW)], tok_v)
    pltpu.sync_copy(cu_hbm.at[pl.ds(0, LANES)], cu_v)

    zero16 = jnp.zeros((LANES,), jnp.float32)

    def _zero(i, carry):
        hist_v[pl.ds(pl.multiple_of(i * LANES, LANES), LANES)] = zero16
        return carry

    lax.fori_loop(0, LANES * NBIN // LANES, _zero, 0)

    cu_r = cu_v[pl.ds(0, LANES)]
    bounds = [_bcast_lane(cu_r, b) for b in range(1, NSEG)]
    iota16 = lax.iota(jnp.int32, LANES)
    lane_off = iota16 * NBIN
    ones16 = jnp.full((LANES,), 1.0, jnp.float32)

    def _scatter(j, carry):
        off = pl.multiple_of(j * LANES, LANES)
        tok = tok_v[pl.ds(off, LANES)]
        idx = base + j * LANES + iota16
        seg = jnp.zeros((LANES,), jnp.int32)
        for bc in bounds:
            seg = seg + (idx >= bc).astype(jnp.int32)
        sidx = lane_off + seg * VPAD + tok
        plsc.addupdate_scatter(hist_v, [sidx], ones16)
        return carry

    lax.fori_loop(0, NVEC, _scatter, 0)

    def _reduce(q, carry):
        qoff = pl.multiple_of(q * LANES, LANES)
        acc = hist_v[pl.ds(qoff, LANES)]
        for l in range(1, LANES):
            acc = acc + hist_v[pl.ds(l * NBIN + qoff, LANES)]
        red_v[pl.ds(qoff, LANES)] = acc
        return carry

    lax.fori_loop(0, NBIN // LANES, _reduce, 0)

    pltpu.sync_copy(red_v, out_hbm.at[w])


_sc_hist = functools.partial(
    pl.kernel,
    out_type=jax.ShapeDtypeStruct((NW, NBIN), jnp.float32),
    mesh=plsc.VectorSubcoreMesh(core_axis_name="c", subcore_axis_name="s"),
    compiler_params=pltpu.CompilerParams(needs_layout_passes=False),
    scratch_types=[
        pltpu.VMEM((TPW,), jnp.int32),
        pltpu.VMEM((LANES,), jnp.int32),
        pltpu.VMEM((LANES * NBIN,), jnp.float32),
        pltpu.VMEM((NBIN,), jnp.float32),
        pltpu.SemaphoreType.DMA,
    ],
)(_sc_hist_body)


def _combine_body(part_ref, tab_ref, w1_ref, b1_ref, w2_ref, b2_ref, out_ref):
    hist = jnp.sum(part_ref[...], axis=0)
    counts = jnp.sum(hist, axis=1, keepdims=True)
    hn = hist[:, :VOCAB] / counts
    h = jax.lax.dot_general(
        tab_ref[...], w1_ref[...], (((1,), (1,)), ((), ())),
        preferred_element_type=jnp.float32,
    ) + b1_ref[...][None, :]
    g = 0.5 * h * (1.0 + jax.lax.erf(h * 0.7071067811865476))
    mo = jax.lax.dot_general(
        g, w2_ref[...], (((1,), (1,)), ((), ())),
        preferred_element_type=jnp.float32,
    ) + b2_ref[...][None, :]
    out_ref[...] = jnp.dot(hn, mo, preferred_element_type=jnp.float32)


def kernel(packed_tokens, cu_seq_lens, table, W1, b1, W2, b2):
    partials = _sc_hist(packed_tokens, cu_seq_lens)
    part3d = partials.reshape(NW, NSEG, VPAD)

    out = pl.pallas_call(
        _combine_body,
        out_shape=jax.ShapeDtypeStruct((NSEG, D), jnp.float32),
    )(part3d, table, W1, b1, W2, b2)
    return out


# speedup vs baseline: 1.0796x; 1.0618x over previous
import functools

import jax
import jax.numpy as jnp
from jax import lax
from jax.experimental import pallas as pl
from jax.experimental.pallas import tpu as pltpu
from jax.experimental.pallas import tpu_sc as plsc

TOTAL = 32768
NSEG = 16
VOCAB = 17
VPAD = 32
D = 512
NC = 2
NS = 16
NW = NC * NS
LANES = 16
TPW = TOTAL // NW
NVEC = TPW // LANES
NBIN = NSEG * VPAD


def _gather_lanes(vec, idx):
    dn = lax.GatherDimensionNumbers(
        offset_dims=(), collapsed_slice_dims=(0,), start_index_map=(0,)
    )
    return lax.gather(
        vec, idx[:, None], dn, (1,),
        mode=lax.GatherScatterMode.PROMISE_IN_BOUNDS,
    )


import os as _os
_SCMODE = _os.environ.get("KPROBE_SC", "")


def _sc_min_body(tok_hbm, cu_hbm, out_hbm, tok_v, cu_v, hist_v, red_v, sem):
    w = lax.axis_index("s") * NC + lax.axis_index("c")
    zero16 = jnp.zeros((LANES,), jnp.float32)

    def _z(i, carry):
        red_v[pl.ds(pl.multiple_of(i * LANES, LANES), LANES)] = zero16
        return carry

    lax.fori_loop(0, NBIN // LANES, _z, 0)
    pltpu.sync_copy(red_v, out_hbm.at[w])


def _sc_hist_body(tok_hbm, cu_hbm, out_hbm, tok_v, cu_v, hist_v, red_v, sem):
    w = lax.axis_index("s") * NC + lax.axis_index("c")
    base = w * TPW
    pltpu.sync_copy(tok_hbm.at[pl.ds(base * 1, TPW)], tok_v)
    pltpu.sync_copy(cu_hbm.at[pl.ds(0, LANES)], cu_v)

    zero16 = jnp.zeros((LANES,), jnp.float32)

    def _zero(i, carry):
        hist_v[pl.ds(pl.multiple_of(i * LANES, LANES), LANES)] = zero16
        return carry

    lax.fori_loop(0, LANES * NBIN // LANES, _zero, 0, unroll=8)

    cu_r = cu_v[pl.ds(0, LANES)]
    iota16 = lax.iota(jnp.int32, LANES)
    lane_off = iota16 * NBIN
    ones16 = jnp.full((LANES,), 1.0, jnp.float32)

    def _scatter(j, idx):
        off = pl.multiple_of(j * LANES, LANES)
        tok = tok_v[pl.ds(off, LANES)]
        seg = jnp.zeros((LANES,), jnp.int32)
        for bit in (8, 4, 2, 1):
            cand = seg + bit
            v = _gather_lanes(cu_r, cand)
            seg = jnp.where(v <= idx, cand, seg)
        sidx = lane_off + seg * VPAD + tok
        plsc.addupdate_scatter(hist_v, [sidx], ones16)
        return idx + LANES

    lax.fori_loop(0, NVEC, _scatter, base + iota16, unroll=4)

    def _reduce(q, carry):
        qoff = pl.multiple_of(q * LANES, LANES)
        acc = hist_v[pl.ds(qoff, LANES)]
        for l in range(1, LANES):
            acc = acc + hist_v[pl.ds(l * NBIN + qoff, LANES)]
        red_v[pl.ds(qoff, LANES)] = acc
        return carry

    lax.fori_loop(0, NBIN // LANES, _reduce, 0, unroll=4)

    pltpu.sync_copy(red_v, out_hbm.at[w])


_sc_hist = functools.partial(
    pl.kernel,
    out_type=jax.ShapeDtypeStruct((NW, NBIN), jnp.float32),
    mesh=plsc.VectorSubcoreMesh(core_axis_name="c", subcore_axis_name="s"),
    compiler_params=pltpu.CompilerParams(needs_layout_passes=False),
    scratch_types=[
        pltpu.VMEM((TPW,), jnp.int32),
        pltpu.VMEM((LANES,), jnp.int32),
        pltpu.VMEM((LANES * NBIN,), jnp.float32),
        pltpu.VMEM((NBIN,), jnp.float32),
        pltpu.SemaphoreType.DMA,
    ],
)(_sc_min_body if _SCMODE == "min" else _sc_hist_body)


def _combine_body(part_ref, tab_ref, w1_ref, b1_ref, w2_ref, b2_ref, out_ref):
    hist = jnp.sum(part_ref[...], axis=0)
    counts = jnp.sum(hist, axis=1, keepdims=True)
    hn = hist[:, :VOCAB] / counts
    h = jax.lax.dot_general(
        tab_ref[...], w1_ref[...], (((1,), (1,)), ((), ())),
        preferred_element_type=jnp.float32,
    ) + b1_ref[...][None, :]
    g = 0.5 * h * (1.0 + jax.lax.erf(h * 0.7071067811865476))
    mo = jax.lax.dot_general(
        g, w2_ref[...], (((1,), (1,)), ((), ())),
        preferred_element_type=jnp.float32,
    ) + b2_ref[...][None, :]
    out_ref[...] = jnp.dot(hn, mo, preferred_element_type=jnp.float32)


def kernel(packed_tokens, cu_seq_lens, table, W1, b1, W2, b2):
    import os as _os
    mode = _os.environ.get("KPROBE", "")
    if mode == "sconly":
        return _sc_hist(packed_tokens, cu_seq_lens)
    if mode == "both":
        partials = _sc_hist(packed_tokens, cu_seq_lens)
        part3d = (packed_tokens[:16384].reshape(NW, NSEG, VPAD)
                  .astype(jnp.float32))
        tcout = pl.pallas_call(
            _combine_body,
            out_shape=jax.ShapeDtypeStruct((NSEG, D), jnp.float32),
        )(part3d, table, W1, b1, W2, b2)
        return partials, tcout
    if mode == "tconly":
        part3d = (packed_tokens[:16384].reshape(NW, NSEG, VPAD)
                  .astype(jnp.float32))
        return pl.pallas_call(
            _combine_body,
            out_shape=jax.ShapeDtypeStruct((NSEG, D), jnp.float32),
        )(part3d, table, W1, b1, W2, b2)
    partials = _sc_hist(packed_tokens, cu_seq_lens)
    part3d = partials.reshape(NW, NSEG, VPAD)

    out = pl.pallas_call(
        _combine_body,
        out_shape=jax.ShapeDtypeStruct((NSEG, D), jnp.float32),
    )(part3d, table, W1, b1, W2, b2)
    return out


# speedup vs baseline: 1.1164x; 1.0341x over previous
import functools

import jax
import jax.numpy as jnp
from jax import lax
from jax.experimental import pallas as pl
from jax.experimental.pallas import tpu as pltpu
from jax.experimental.pallas import tpu_sc as plsc

TOTAL = 32768
NSEG = 16
VOCAB = 17
VPAD = 32
D = 512
NC = 2
NS = 16
NW = NC * NS
LANES = 16
TPW = TOTAL // NW
NVEC = TPW // LANES
NBIN = NSEG * VPAD


def _gather_lanes(vec, idx):
    dn = lax.GatherDimensionNumbers(
        offset_dims=(), collapsed_slice_dims=(0,), start_index_map=(0,)
    )
    return lax.gather(
        vec, idx[:, None], dn, (1,),
        mode=lax.GatherScatterMode.PROMISE_IN_BOUNDS,
    )


def _sc_hist_body(tok_hbm, cu_hbm, out_hbm, tok_v, cu_v, hist_v, red_v, sem):
    w = lax.axis_index("s") * NC + lax.axis_index("c")
    base = w * TPW
    pltpu.sync_copy(tok_hbm.at[pl.ds(base * 1, TPW)], tok_v)
    pltpu.sync_copy(cu_hbm.at[pl.ds(0, LANES)], cu_v)

    zero16 = jnp.zeros((LANES,), jnp.float32)

    def _zero(i, carry):
        hist_v[pl.ds(pl.multiple_of(i * LANES, LANES), LANES)] = zero16
        return carry

    lax.fori_loop(0, LANES * NBIN // LANES, _zero, 0, unroll=8)

    cu_r = cu_v[pl.ds(0, LANES)]
    iota16 = lax.iota(jnp.int32, LANES)
    lane_off = iota16 * NBIN
    ones16 = jnp.full((LANES,), 1.0, jnp.float32)

    def _scatter(j, idx):
        off = pl.multiple_of(j * LANES, LANES)
        tok = tok_v[pl.ds(off, LANES)]
        seg = jnp.zeros((LANES,), jnp.int32)
        for bit in (8, 4, 2, 1):
            cand = seg + bit
            v = _gather_lanes(cu_r, cand)
            seg = jnp.where(v <= idx, cand, seg)
        sidx = lane_off + seg * VPAD + tok
        plsc.addupdate_scatter(hist_v, [sidx], ones16)
        return idx + LANES

    lax.fori_loop(0, NVEC, _scatter, base + iota16, unroll=4)

    def _reduce(q, carry):
        qoff = pl.multiple_of(q * LANES, LANES)
        acc = hist_v[pl.ds(qoff, LANES)]
        for l in range(1, LANES):
            acc = acc + hist_v[pl.ds(l * NBIN + qoff, LANES)]
        red_v[pl.ds(qoff, LANES)] = acc
        return carry

    lax.fori_loop(0, NBIN // LANES, _reduce, 0, unroll=4)

    pltpu.sync_copy(red_v, out_hbm.at[w])


_sc_hist = functools.partial(
    pl.kernel,
    out_type=jax.ShapeDtypeStruct((NW, NBIN), jnp.float32),
    mesh=plsc.VectorSubcoreMesh(
        core_axis_name="c", subcore_axis_name="s",
        num_cores=NC, num_subcores=NS,
    ),
    compiler_params=pltpu.CompilerParams(needs_layout_passes=False),
    scratch_types=[
        pltpu.VMEM((TPW,), jnp.int32),
        pltpu.VMEM((LANES,), jnp.int32),
        pltpu.VMEM((LANES * NBIN,), jnp.float32),
        pltpu.VMEM((NBIN,), jnp.float32),
        pltpu.SemaphoreType.DMA,
    ],
)(_sc_hist_body)


def _mlp_body(tab_ref, w1_ref, b1_ref, w2_ref, b2_ref, out_ref):
    h = jax.lax.dot_general(
        tab_ref[...], w1_ref[...], (((1,), (1,)), ((), ())),
        preferred_element_type=jnp.float32,
    ) + b1_ref[...][None, :]
    g = 0.5 * h * (1.0 + jax.lax.erf(h * 0.7071067811865476))
    out_ref[...] = jax.lax.dot_general(
        g, w2_ref[...], (((1,), (1,)), ((), ())),
        preferred_element_type=jnp.float32,
    ) + b2_ref[...][None, :]


def _combine_body(part_ref, mlp_ref, out_ref):
    hist = jnp.sum(part_ref[...], axis=0)
    counts = jnp.sum(hist, axis=1, keepdims=True)
    hn = hist[:, :VOCAB] / counts
    out_ref[...] = jnp.dot(hn, mlp_ref[...], preferred_element_type=jnp.float32)


def kernel(packed_tokens, cu_seq_lens, table, W1, b1, W2, b2):
    partials = _sc_hist(packed_tokens, cu_seq_lens)
    mlp = pl.pallas_call(
        _mlp_body,
        out_shape=jax.ShapeDtypeStruct((VOCAB, D), jnp.float32),
    )(table, W1, b1, W2, b2)

    part3d = partials.reshape(NW, NSEG, VPAD)
    out = pl.pallas_call(
        _combine_body,
        out_shape=jax.ShapeDtypeStruct((NSEG, D), jnp.float32),
    )(part3d, mlp)
    return out


# speedup vs baseline: 1.1841x; 1.0606x over previous
import functools

import jax
import jax.numpy as jnp
from jax import lax
from jax.experimental import pallas as pl
from jax.experimental.pallas import tpu as pltpu
from jax.experimental.pallas import tpu_sc as plsc

TOTAL = 32768
NSEG = 16
VOCAB = 17
VPAD = 32
D = 512
NC = 2
NS = 16
NW = NC * NS
LANES = 16
TPW = TOTAL // NW
NVEC = TPW // LANES
NBIN = NSEG * VPAD


def _gather_lanes(vec, idx):
    dn = lax.GatherDimensionNumbers(
        offset_dims=(), collapsed_slice_dims=(0,), start_index_map=(0,)
    )
    return lax.gather(
        vec, idx[:, None], dn, (1,),
        mode=lax.GatherScatterMode.PROMISE_IN_BOUNDS,
    )


def _sc_hist_body(tok_hbm, cu_hbm, out_hbm, tok_v, cu_v, hist_v, red_v, sem):
    w = lax.axis_index("s") * NC + lax.axis_index("c")
    base = w * TPW
    pltpu.sync_copy(tok_hbm.at[pl.ds(base * 1, TPW)], tok_v)
    pltpu.sync_copy(cu_hbm.at[pl.ds(0, LANES)], cu_v)

    zero16 = jnp.zeros((LANES,), jnp.float32)

    def _zero(i, carry):
        hist_v[pl.ds(pl.multiple_of(i * LANES, LANES), LANES)] = zero16
        return carry

    lax.fori_loop(0, LANES * NBIN // LANES, _zero, 0, unroll=8)

    cu_r = cu_v[pl.ds(0, LANES)]
    iota16 = lax.iota(jnp.int32, LANES)
    lane_off = iota16 * NBIN
    ones16 = jnp.full((LANES,), 1.0, jnp.float32)

    def _scatter(j, idx):
        off = pl.multiple_of(j * LANES, LANES)
        tok = tok_v[pl.ds(off, LANES)]
        seg = jnp.zeros((LANES,), jnp.int32)
        for bit in (8, 4, 2, 1):
            cand = seg + bit
            v = _gather_lanes(cu_r, cand)
            seg = jnp.where(v <= idx, cand, seg)
        sidx = lane_off + seg * VPAD + tok
        plsc.addupdate_scatter(hist_v, [sidx], ones16)
        return idx + LANES

    lax.fori_loop(0, NVEC, _scatter, base + iota16, unroll=4)

    def _reduce(q, carry):
        qoff = pl.multiple_of(q * LANES, LANES)
        acc = hist_v[pl.ds(qoff, LANES)]
        for l in range(1, LANES):
            acc = acc + hist_v[pl.ds(l * NBIN + qoff, LANES)]
        red_v[q // 2, pl.ds(pl.multiple_of((q % 2) * LANES, LANES), LANES)] = acc
        return carry

    lax.fori_loop(0, NBIN // LANES, _reduce, 0, unroll=4)

    pltpu.sync_copy(red_v, out_hbm.at[w])


_sc_hist = functools.partial(
    pl.kernel,
    out_type=jax.ShapeDtypeStruct((NW, NSEG, VPAD), jnp.float32),
    mesh=plsc.VectorSubcoreMesh(
        core_axis_name="c", subcore_axis_name="s",
        num_cores=NC, num_subcores=NS,
    ),
    compiler_params=pltpu.CompilerParams(needs_layout_passes=False),
    scratch_types=[
        pltpu.VMEM((TPW,), jnp.int32),
        pltpu.VMEM((LANES,), jnp.int32),
        pltpu.VMEM((LANES * NBIN,), jnp.float32),
        pltpu.VMEM((NSEG, VPAD), jnp.float32),
        pltpu.SemaphoreType.DMA,
    ],
)(_sc_hist_body)


def _mlp_body(tab_ref, w1_ref, b1_ref, w2_ref, b2_ref, out_ref):
    h = jax.lax.dot_general(
        tab_ref[...], w1_ref[...], (((1,), (1,)), ((), ())),
        preferred_element_type=jnp.float32,
    ) + b1_ref[...][None, :]
    g = 0.5 * h * (1.0 + jax.lax.erf(h * 0.7071067811865476))
    out_ref[...] = jax.lax.dot_general(
        g, w2_ref[...], (((1,), (1,)), ((), ())),
        preferred_element_type=jnp.float32,
    ) + b2_ref[...][None, :]


def _combine_body(part_ref, mlp_ref, out_ref):
    hist = jnp.sum(part_ref[...], axis=0)
    counts = jnp.sum(hist, axis=1, keepdims=True)
    hn = hist[:, :VOCAB] / counts
    out_ref[...] = jnp.dot(hn, mlp_ref[...], preferred_element_type=jnp.float32)


def kernel(packed_tokens, cu_seq_lens, table, W1, b1, W2, b2):
    part3d = _sc_hist(packed_tokens, cu_seq_lens)
    mlp = pl.pallas_call(
        _mlp_body,
        out_shape=jax.ShapeDtypeStruct((VOCAB, D), jnp.float32),
    )(table, W1, b1, W2, b2)

    out = pl.pallas_call(
        _combine_body,
        out_shape=jax.ShapeDtypeStruct((NSEG, D), jnp.float32),
    )(part3d, mlp)
    return out


# speedup vs baseline: 1.2163x; 1.0272x over previous
import functools

import jax
import jax.numpy as jnp
from jax import lax
from jax.experimental import pallas as pl
from jax.experimental.pallas import tpu as pltpu
from jax.experimental.pallas import tpu_sc as plsc

TOTAL = 32768
NSEG = 16
VOCAB = 17
VPAD = 32
D = 512
NC = 2
NS = 16
NW = NC * NS
LANES = 16
TPW = TOTAL // NW
NVEC = TPW // LANES
NBIN = NSEG * VPAD


def _gather_lanes(vec, idx):
    dn = lax.GatherDimensionNumbers(
        offset_dims=(), collapsed_slice_dims=(0,), start_index_map=(0,)
    )
    return lax.gather(
        vec, idx[:, None], dn, (1,),
        mode=lax.GatherScatterMode.PROMISE_IN_BOUNDS,
    )


def _sc_hist_body(tok_hbm, cu_hbm, out_hbm, tok_v, cu_v, hist_v, red_v, sem):
    w = lax.axis_index("s") * NC + lax.axis_index("c")
    base = w * TPW
    cp = pltpu.async_copy(tok_hbm.at[pl.ds(base * 1, TPW)], tok_v, sem)
    pltpu.sync_copy(cu_hbm.at[pl.ds(0, LANES)], cu_v)

    zero16 = jnp.zeros((LANES,), jnp.float32)

    def _zero(i, carry):
        hist_v[pl.ds(pl.multiple_of(i * LANES, LANES), LANES)] = zero16
        return carry

    lax.fori_loop(0, LANES * NBIN // LANES, _zero, 0, unroll=16)
    cp.wait()

    cu_r = cu_v[pl.ds(0, LANES)]
    iota16 = lax.iota(jnp.int32, LANES)
    lane_off = iota16 * NBIN
    ones16 = jnp.full((LANES,), 1.0, jnp.float32)

    def _scatter(j, idx):
        off = pl.multiple_of(j * LANES, LANES)
        tok = tok_v[pl.ds(off, LANES)]
        seg = jnp.zeros((LANES,), jnp.int32)
        for bit in (8, 4, 2, 1):
            cand = seg + bit
            v = _gather_lanes(cu_r, cand)
            seg = jnp.where(v <= idx, cand, seg)
        sidx = lane_off + seg * VPAD + tok
        plsc.addupdate_scatter(hist_v, [sidx], ones16)
        return idx + LANES

    lax.fori_loop(0, NVEC, _scatter, base + iota16, unroll=8)

    def _reduce(q, carry):
        qoff = pl.multiple_of(q * LANES, LANES)
        acc = hist_v[pl.ds(qoff, LANES)]
        for l in range(1, LANES):
            acc = acc + hist_v[pl.ds(l * NBIN + qoff, LANES)]
        red_v[q // 2, pl.ds(pl.multiple_of((q % 2) * LANES, LANES), LANES)] = acc
        return carry

    lax.fori_loop(0, NBIN // LANES, _reduce, 0, unroll=4)

    pltpu.sync_copy(red_v, out_hbm.at[w])


_sc_hist = functools.partial(
    pl.kernel,
    out_type=jax.ShapeDtypeStruct((NW, NSEG, VPAD), jnp.float32),
    mesh=plsc.VectorSubcoreMesh(
        core_axis_name="c", subcore_axis_name="s",
        num_cores=NC, num_subcores=NS,
    ),
    compiler_params=pltpu.CompilerParams(needs_layout_passes=False),
    scratch_types=[
        pltpu.VMEM((TPW,), jnp.int32),
        pltpu.VMEM((LANES,), jnp.int32),
        pltpu.VMEM((LANES * NBIN,), jnp.float32),
        pltpu.VMEM((NSEG, VPAD), jnp.float32),
        pltpu.SemaphoreType.DMA,
    ],
)(_sc_hist_body)


def _mlp_body(tab_ref, w1_ref, b1_ref, w2_ref, b2_ref, out_ref):
    h = jax.lax.dot_general(
        tab_ref[...], w1_ref[...], (((1,), (1,)), ((), ())),
        preferred_element_type=jnp.float32,
    ) + b1_ref[...][None, :]
    g = 0.5 * h * (1.0 + jax.lax.erf(h * 0.7071067811865476))
    out_ref[...] = jax.lax.dot_general(
        g, w2_ref[...], (((1,), (1,)), ((), ())),
        preferred_element_type=jnp.float32,
    ) + b2_ref[...][None, :]


def _combine_body(part_ref, mlp_ref, out_ref):
    hist = jnp.sum(part_ref[...], axis=0)
    counts = jnp.sum(hist, axis=1, keepdims=True)
    hn = hist[:, :VOCAB] / counts
    out_ref[...] = jnp.dot(hn, mlp_ref[...], preferred_element_type=jnp.float32)


def kernel(packed_tokens, cu_seq_lens, table, W1, b1, W2, b2):
    part3d = _sc_hist(packed_tokens, cu_seq_lens)
    mlp = pl.pallas_call(
        _mlp_body,
        out_shape=jax.ShapeDtypeStruct((VOCAB, D), jnp.float32),
    )(table, W1, b1, W2, b2)

    out = pl.pallas_call(
        _combine_body,
        out_shape=jax.ShapeDtypeStruct((NSEG, D), jnp.float32),
    )(part3d, mlp)
    return out


# speedup vs baseline: 1.2166x; 1.0002x over previous
import functools

import jax
import jax.numpy as jnp
from jax import lax
from jax.experimental import pallas as pl
from jax.experimental.pallas import tpu as pltpu
from jax.experimental.pallas import tpu_sc as plsc

TOTAL = 32768
NSEG = 16
VOCAB = 17
VPAD = 32
D = 512
NC = 2
NS = 16
NW = NC * NS
LANES = 16
TPW = TOTAL // NW
NVEC = TPW // LANES
NBIN = NSEG * VPAD


def _gather_lanes(vec, idx):
    dn = lax.GatherDimensionNumbers(
        offset_dims=(), collapsed_slice_dims=(0,), start_index_map=(0,)
    )
    return lax.gather(
        vec, idx[:, None], dn, (1,),
        mode=lax.GatherScatterMode.PROMISE_IN_BOUNDS,
    )


def _sc_hist_body(tok_hbm, cu_hbm, out_hbm, tok_v, cu_v, hist_v, red_v, sem):
    w = lax.axis_index("s") * NC + lax.axis_index("c")
    base = w * TPW
    cp = pltpu.async_copy(tok_hbm.at[pl.ds(base, TPW)], tok_v, sem)
    pltpu.sync_copy(cu_hbm.at[pl.ds(0, LANES)], cu_v)

    zero16 = jnp.zeros((LANES,), jnp.float32)

    def _zero(i, carry):
        hist_v[pl.ds(pl.multiple_of(i * LANES, LANES), LANES)] = zero16
        return carry

    lax.fori_loop(0, LANES * NBIN // LANES, _zero, 0, unroll=16)
    cp.wait()

    cu_r = cu_v[pl.ds(0, LANES)]
    iota16 = lax.iota(jnp.int32, LANES)
    lane_off = iota16 * NBIN
    ones16 = jnp.full((LANES,), 1.0, jnp.float32)

    def _scatter(j, idx):
        off = pl.multiple_of(j * LANES, LANES)
        tok = tok_v[pl.ds(off, LANES)]
        seg = jnp.zeros((LANES,), jnp.int32)
        for bit in (8, 4, 2, 1):
            cand = seg + bit
            v = _gather_lanes(cu_r, cand)
            seg = jnp.where(v <= idx, cand, seg)
        sidx = lane_off + seg * VPAD + tok
        plsc.addupdate_scatter(hist_v, [sidx], ones16)
        return idx + LANES

    lax.fori_loop(0, NVEC, _scatter, base + iota16, unroll=8)

    def _reduce(q, carry):
        qoff = pl.multiple_of(q * LANES, LANES)
        acc = hist_v[pl.ds(qoff, LANES)]
        for l in range(1, LANES):
            acc = acc + hist_v[pl.ds(l * NBIN + qoff, LANES)]
        red_v[q // 2, pl.ds(pl.multiple_of((q % 2) * LANES, LANES), LANES)] = acc
        return carry

    lax.fori_loop(0, NBIN // LANES, _reduce, 0, unroll=4)

    pltpu.sync_copy(red_v, out_hbm.at[w])


_sc_hist = functools.partial(
    pl.kernel,
    out_type=jax.ShapeDtypeStruct((NW, NSEG, VPAD), jnp.float32),
    mesh=plsc.VectorSubcoreMesh(
        core_axis_name="c", subcore_axis_name="s",
        num_cores=NC, num_subcores=NS,
    ),
    compiler_params=pltpu.CompilerParams(needs_layout_passes=False),
    scratch_types=[
        pltpu.VMEM((TPW,), jnp.int32),
        pltpu.VMEM((LANES,), jnp.int32),
        pltpu.VMEM((LANES * NBIN,), jnp.float32),
        pltpu.VMEM((NSEG, VPAD), jnp.float32),
        pltpu.SemaphoreType.DMA,
    ],
)(_sc_hist_body)


def _mlp_body(tab_ref, w1_ref, b1_ref, w2_ref, b2_ref, out_ref):
    h = jax.lax.dot_general(
        tab_ref[...], w1_ref[...], (((1,), (1,)), ((), ())),
        preferred_element_type=jnp.float32,
    ) + b1_ref[...][None, :]
    g = 0.5 * h * (1.0 + jax.lax.erf(h * 0.7071067811865476))
    out_ref[...] = jax.lax.dot_general(
        g, w2_ref[...], (((1,), (1,)), ((), ())),
        preferred_element_type=jnp.float32,
    ) + b2_ref[...][None, :]


def _combine_body(part_ref, mlp_ref, out_ref):
    hist = jnp.sum(part_ref[...], axis=0)
    counts = jnp.sum(hist, axis=1, keepdims=True)
    hn = hist[:, :VOCAB] / counts
    out_ref[...] = jnp.dot(hn, mlp_ref[...], preferred_element_type=jnp.float32)


def kernel(packed_tokens, cu_seq_lens, table, W1, b1, W2, b2):
    part3d = _sc_hist(packed_tokens, cu_seq_lens)
    mlp = pl.pallas_call(
        _mlp_body,
        out_shape=jax.ShapeDtypeStruct((VOCAB, D), jnp.float32),
    )(table, W1, b1, W2, b2)

    out = pl.pallas_call(
        _combine_body,
        out_shape=jax.ShapeDtypeStruct((NSEG, D), jnp.float32),
    )(part3d, mlp)
    return out
